# baseline (speedup 1.0000x reference)
import jax
import jax.numpy as jnp
from jax.experimental import pallas as pl

N = 2048
D = 256
K = 512
NB = 512
GRID = N // NB


def _kmeans_kernel(x_ref, y_ref, c_ref, ynew_ref, loss_ref):
    x = x_ref[...]
    c = c_ref[...]
    yb = y_ref[...]

    st = jax.lax.dot_general(
        c + c, x,
        dimension_numbers=(((1,), (1,)), ((), ())),
        preferred_element_type=jnp.float32,
        precision=jax.lax.Precision.HIGHEST,
    )
    c2 = jnp.sum(c * c, axis=1, keepdims=True)
    s = c2 - st

    smin = jnp.min(s, axis=0, keepdims=True)
    row = jax.lax.broadcasted_iota(jnp.int32, (K, NB), 0)
    ynew_ref[...] = jnp.min(jnp.where(s == smin, row, K), axis=0, keepdims=True)

    hit = jnp.where(row == yb, s, 0.0)
    part = (jnp.sum(x * x, axis=(0, 1), keepdims=True)
            + jnp.sum(hit, axis=(0, 1), keepdims=True))

    @pl.when(pl.program_id(0) == 0)
    def _init():
        loss_ref[...] = jnp.zeros((1, 1), jnp.float32)

    loss_ref[...] += part


def kernel(x, y, centers):
    y2 = y.reshape(1, N)
    ynew2, loss2 = pl.pallas_call(
        _kmeans_kernel,
        grid=(GRID,),
        in_specs=[
            pl.BlockSpec((NB, D), lambda i: (i, 0)),
            pl.BlockSpec((1, NB), lambda i: (0, i)),
            pl.BlockSpec((K, D), lambda i: (0, 0)),
        ],
        out_specs=(
            pl.BlockSpec((1, NB), lambda i: (0, i)),
            pl.BlockSpec((1, 1), lambda i: (0, 0)),
        ),
        out_shape=(
            jax.ShapeDtypeStruct((1, N), jnp.int32),
            jax.ShapeDtypeStruct((1, 1), jnp.float32),
        ),
    )(x, y2, centers)
    return (loss2[0, 0], ynew2.reshape(N))


# speedup vs baseline: 1.0052x; 1.0052x over previous
import jax
import jax.numpy as jnp
from jax.experimental import pallas as pl

N = 2048
D = 256
K = 512
NB = 1024
GRID = N // NB


def _kmeans_kernel(x_ref, y_ref, c_ref, ynew_ref, loss_ref):
    x = x_ref[...]
    c = c_ref[...]
    yb = y_ref[...]

    st = jax.lax.dot_general(
        c + c, x,
        dimension_numbers=(((1,), (1,)), ((), ())),
        preferred_element_type=jnp.float32,
        precision=jax.lax.Precision.HIGHEST,
    )
    c2 = jnp.sum(c * c, axis=1, keepdims=True)
    s = c2 - st

    smin = jnp.min(s, axis=0, keepdims=True)
    row = jax.lax.broadcasted_iota(jnp.int32, (K, NB), 0)
    ynew_ref[...] = jnp.min(jnp.where(s == smin, row, K), axis=0, keepdims=True)

    hit = jnp.where(row == yb, s, 0.0)
    part = (jnp.sum(x * x, axis=(0, 1), keepdims=True)
            + jnp.sum(hit, axis=(0, 1), keepdims=True))

    @pl.when(pl.program_id(0) == 0)
    def _init():
        loss_ref[...] = jnp.zeros((1, 1), jnp.float32)

    loss_ref[...] += part


def kernel(x, y, centers):
    y2 = y.reshape(1, N)
    ynew2, loss2 = pl.pallas_call(
        _kmeans_kernel,
        grid=(GRID,),
        in_specs=[
            pl.BlockSpec((NB, D), lambda i: (i, 0)),
            pl.BlockSpec((1, NB), lambda i: (0, i)),
            pl.BlockSpec((K, D), lambda i: (0, 0)),
        ],
        out_specs=(
            pl.BlockSpec((1, NB), lambda i: (0, i)),
            pl.BlockSpec((1, 1), lambda i: (0, 0)),
        ),
        out_shape=(
            jax.ShapeDtypeStruct((1, N), jnp.int32),
            jax.ShapeDtypeStruct((1, 1), jnp.float32),
        ),
    )(x, y2, centers)
    return (loss2[0, 0], ynew2.reshape(N))


# speedup vs baseline: 1.0234x; 1.0181x over previous
import jax
import jax.numpy as jnp
from jax.experimental import pallas as pl
from jax.experimental.pallas import tpu as pltpu

N = 2048
D = 256
K = 512
NB = 512
GRID = N // NB


def _kmeans_kernel(x_ref, y_ref, c_ref, ynew_ref, loss_ref, c2x_ref, c2_ref):
    i = pl.program_id(0)

    @pl.when(i == 0)
    def _init():
        c = c_ref[...]
        c2x_ref[...] = c + c
        c2_ref[...] = jnp.sum(c * c, axis=1, keepdims=True)
        loss_ref[...] = jnp.zeros((1, 1), jnp.float32)

    x = x_ref[...]
    yb = y_ref[...]

    st = jax.lax.dot_general(
        c2x_ref[...], x,
        dimension_numbers=(((1,), (1,)), ((), ())),
        preferred_element_type=jnp.float32,
        precision=jax.lax.Precision.HIGHEST,
    )
    s = c2_ref[...] - st

    ynew_ref[...] = jnp.argmin(s, axis=0, keepdims=True).astype(jnp.int32)

    row = jax.lax.broadcasted_iota(jnp.int32, (K, NB), 0)
    hit = jnp.where(row == yb, s, 0.0)
    loss_ref[...] += (jnp.sum(x * x, axis=(0, 1), keepdims=True)
                      + jnp.sum(hit, axis=(0, 1), keepdims=True))


def kernel(x, y, centers):
    y2 = y.reshape(1, N)
    ynew2, loss2 = pl.pallas_call(
        _kmeans_kernel,
        grid=(GRID,),
        in_specs=[
            pl.BlockSpec((NB, D), lambda i: (i, 0)),
            pl.BlockSpec((1, NB), lambda i: (0, i)),
            pl.BlockSpec((K, D), lambda i: (0, 0)),
        ],
        out_specs=(
            pl.BlockSpec((1, NB), lambda i: (0, i)),
            pl.BlockSpec((1, 1), lambda i: (0, 0)),
        ),
        out_shape=(
            jax.ShapeDtypeStruct((1, N), jnp.int32),
            jax.ShapeDtypeStruct((1, 1), jnp.float32),
        ),
        scratch_shapes=[
            pltpu.VMEM((K, D), jnp.float32),
            pltpu.VMEM((K, 1), jnp.float32),
        ],
    )(x, y2, centers)
    return (loss2[0, 0], ynew2.reshape(N))


# speedup vs baseline: 1.0399x; 1.0161x over previous
import jax
import jax.numpy as jnp
from jax.experimental import pallas as pl

N = 2048
D = 256
K = 512


def _kmeans_kernel(x_ref, y_ref, c_ref, ynew_ref, loss_ref):
    x = x_ref[...]
    c = c_ref[...]
    yb = y_ref[...]

    st = jax.lax.dot_general(
        c + c, x,
        dimension_numbers=(((1,), (1,)), ((), ())),
        preferred_element_type=jnp.float32,
        precision=jax.lax.Precision.HIGHEST,
    )
    c2 = jnp.sum(c * c, axis=1, keepdims=True)
    s = c2 - st

    smin = jnp.min(s, axis=0, keepdims=True)
    row = jax.lax.broadcasted_iota(jnp.int32, (K, N), 0)
    ynew_ref[...] = jnp.min(jnp.where(s == smin, row, K), axis=0, keepdims=True)

    hit = jnp.where(row == yb, s, 0.0)
    loss_ref[...] = (jnp.sum(x * x, axis=(0, 1), keepdims=True)
                     + jnp.sum(hit, axis=(0, 1), keepdims=True))


def kernel(x, y, centers):
    y2 = y.reshape(1, N)
    ynew2, loss2 = pl.pallas_call(
        _kmeans_kernel,
        out_shape=(
            jax.ShapeDtypeStruct((1, N), jnp.int32),
            jax.ShapeDtypeStruct((1, 1), jnp.float32),
        ),
    )(x, y2, centers)
    return (loss2[0, 0], ynew2.reshape(N))
